# baseline (speedup 1.0000x reference)
import functools
import jax
import jax.numpy as jnp
from jax import lax
from jax.experimental import pallas as pl
from jax.experimental.pallas import tpu as pltpu
from jax.experimental.pallas import tpu_sc as plsc

D_MODEL = 768
N_PATCHES = 1024
N_TOT = N_PATCHES + 1
BATCH = 64
NW = 32
GROUPS = N_PATCHES // 8
GPW = GROUPS // NW
BC = 8
NBC = BATCH // BC
NV = D_MODEL // 16


def _sc_body(in_hbm, cls_hbm, pos_hbm, out_hbm, ibuf, obuf, posb, clsb):
    c = lax.axis_index("c")
    s = lax.axis_index("s")
    wid = s * 2 + c

    @pl.when(wid == 0)
    def _plane0():
        pltpu.sync_copy(cls_hbm, clsb)
        pltpu.sync_copy(pos_hbm.at[pl.ds(0, 16), :], posb)
        for b in range(8):
            for k in range(NV):
                sl = pl.ds(16 * k, 16)
                clsb[b, sl] = clsb[b, sl] + posb[0, sl]

        def wr(cb, _):
            pltpu.sync_copy(clsb, out_hbm.at[0, pl.ds(8 * cb, 8), :])
            return 0

        lax.fori_loop(0, BATCH // 8, wr, 0)

    def per_group(g, _):
        j = wid * GPW + g
        pltpu.sync_copy(pos_hbm.at[pl.ds(8 * j, 16), :], posb)

        def per_chunk(ci, _):
            b0 = ci * BC
            pltpu.sync_copy(
                in_hbm.at[pl.ds(b0, BC), pl.ds(8 * j, 8), :], ibuf
            )

            def per_r(r, _):
                for b in range(BC):
                    for k in range(NV):
                        sl = pl.ds(16 * k, 16)
                        obuf[r, b, sl] = ibuf[b, r, sl] + posb[r + 1, sl]
                return 0

            lax.fori_loop(0, 8, per_r, 0)
            pltpu.sync_copy(
                obuf, out_hbm.at[pl.ds(8 * j + 1, 8), pl.ds(b0, BC), :]
            )
            return 0

        lax.fori_loop(0, NBC, per_chunk, 0)
        return 0

    lax.fori_loop(0, GPW, per_group, 0)


def kernel(inputs, class_embed, pos_table):
    mesh = plsc.VectorSubcoreMesh(core_axis_name="c", subcore_axis_name="s")
    f = pl.kernel(
        _sc_body,
        mesh=mesh,
        out_type=jax.ShapeDtypeStruct((N_TOT, BATCH, D_MODEL), jnp.float32),
        scratch_types=[
            pltpu.VMEM((BC, 8, D_MODEL), jnp.float32),
            pltpu.VMEM((8, BC, D_MODEL), jnp.float32),
            pltpu.VMEM((16, D_MODEL), jnp.float32),
            pltpu.VMEM((8, D_MODEL), jnp.float32),
        ],
    )
    cls8 = jnp.broadcast_to(class_embed.reshape(1, D_MODEL), (8, D_MODEL))
    pos_pad = jnp.pad(pos_table, ((0, 7), (0, 0)))
    res = f(inputs, cls8, pos_pad)
    return jnp.transpose(res, (1, 0, 2))


# speedup vs baseline: 6.5407x; 6.5407x over previous
import jax
import jax.numpy as jnp
from jax.experimental import pallas as pl
from jax.experimental.pallas import tpu as pltpu

D_MODEL = 768
N_PATCHES = 1024
N_TOT = N_PATCHES + 1
BATCH = 64
Q = 64
NSTEP = (N_TOT + Q - 1) // Q


def _body(in_ref, cls_ref, pos_ref, out_ref, prev_ref):
    j = pl.program_id(0)

    @pl.when(j == 0)
    def _cls():
        out_ref[0] = jnp.broadcast_to(
            cls_ref[0, 0, :] + pos_ref[0], (BATCH, D_MODEL)
        )

    @pl.when(j > 0)
    def _carry():
        out_ref[0] = prev_ref[...] + pos_ref[0][None, :]

    for r in range(1, Q):
        out_ref[r] = in_ref[:, r - 1, :] + pos_ref[r][None, :]
    prev_ref[...] = in_ref[:, Q - 1, :]


def kernel(inputs, class_embed, pos_table):
    res = pl.pallas_call(
        _body,
        grid=(NSTEP,),
        in_specs=[
            pl.BlockSpec(
                (BATCH, Q, D_MODEL),
                lambda j: (0, jnp.minimum(j, N_PATCHES // Q - 1), 0),
            ),
            pl.BlockSpec((1, 1, D_MODEL), lambda j: (0, 0, 0)),
            pl.BlockSpec((Q, D_MODEL), lambda j: (j, 0)),
        ],
        out_specs=pl.BlockSpec((Q, BATCH, D_MODEL), lambda j: (j, 0, 0)),
        out_shape=jax.ShapeDtypeStruct((N_TOT, BATCH, D_MODEL), jnp.float32),
        scratch_shapes=[pltpu.VMEM((BATCH, D_MODEL), jnp.float32)],
    )(inputs, class_embed, pos_table)
    return jnp.transpose(res, (1, 0, 2))
